# baseline (speedup 1.0000x reference)
import functools

import jax
import jax.numpy as jnp
from jax import lax
from jax.experimental import pallas as pl
from jax.experimental.pallas import tpu as pltpu
from jax.experimental.pallas import tpu_sc as plsc

VOCAB = 1000
VPAD = 1024
B = 1024
T = 20
TPAD = 24
NC, NS = 2, 16
NW = NC * NS
B_PER_W = B // NW
PAIRS = B_PER_W // 2

_mesh = plsc.VectorSubcoreMesh(core_axis_name="c", subcore_axis_name="s")


@functools.partial(
    pl.kernel,
    out_type=jax.ShapeDtypeStruct((B * TPAD, VPAD), jnp.float32),
    mesh=_mesh,
    scratch_types=[
        pltpu.VMEM((B_PER_W * TPAD,), jnp.int32),
        pltpu.VMEM((2 * TPAD, VPAD), jnp.float32),
        pltpu.VMEM((2 * TPAD, VPAD), jnp.float32),
        pltpu.SemaphoreType.DMA,
        pltpu.SemaphoreType.DMA,
    ],
)
def _gather_rows(emb_hbm, idx_hbm, out_hbm, idx_v, buf0, buf1, sem0, sem1):
    wid = lax.axis_index("s") * NC + lax.axis_index("c")
    base_b = wid * B_PER_W
    pltpu.sync_copy(idx_hbm.at[pl.ds(base_b * TPAD, B_PER_W * TPAD)], idx_v)

    bufs = (buf0, buf1)
    sems = (sem0, sem1)

    def start_gather(j):
        cp = pltpu.make_async_copy(
            emb_hbm.at[idx_v.at[pl.ds(j * 2 * TPAD, 2 * TPAD)]],
            bufs[j % 2],
            sems[j % 2],
        )
        cp.start()
        return cp

    copies = [start_gather(0)]
    for j in range(PAIRS):
        if j + 1 < PAIRS:
            copies.append(start_gather(j + 1))
        copies[j].wait()
        pltpu.sync_copy(bufs[j % 2],
                        out_hbm.at[pl.ds((base_b + 2 * j) * TPAD, 2 * TPAD)])


def kernel(xb, emb):
    embp = jnp.pad(emb, ((0, 0), (0, VPAD - VOCAB)))
    idx = jnp.pad(xb, ((0, 0), (0, TPAD - T))).reshape(-1)
    out = _gather_rows(embp, idx)
    return out.reshape(B, TPAD, VPAD)[:, :T, :VOCAB]


# speedup vs baseline: 1.7794x; 1.7794x over previous
import functools

import jax
import jax.numpy as jnp
from jax import lax
from jax.experimental import pallas as pl
from jax.experimental.pallas import tpu as pltpu
from jax.experimental.pallas import tpu_sc as plsc

VOCAB = 1000
B = 1024
T = 20
NC, NS = 2, 16
NW = NC * NS
B_PER_W = B // NW
BPC = 2
CHUNK = BPC * T
N_CHUNKS = B_PER_W // BPC

_mesh = plsc.VectorSubcoreMesh(core_axis_name="c", subcore_axis_name="s")


@functools.partial(
    pl.kernel,
    out_type=jax.ShapeDtypeStruct((B, T, VOCAB), jnp.float32),
    mesh=_mesh,
    compiler_params=pltpu.CompilerParams(use_tc_tiling_on_sc=False),
    scratch_types=[
        pltpu.VMEM((B_PER_W * T,), jnp.int32),
        pltpu.VMEM((CHUNK, VOCAB), jnp.float32),
        pltpu.VMEM((CHUNK, VOCAB), jnp.float32),
        pltpu.SemaphoreType.DMA,
        pltpu.SemaphoreType.DMA,
    ],
)
def _gather_rows(emb_hbm, idx_hbm, out_hbm, idx_v, buf0, buf1, sem0, sem1):
    wid = lax.axis_index("s") * NC + lax.axis_index("c")
    base_b = wid * B_PER_W
    pltpu.sync_copy(idx_hbm.at[pl.ds(base_b * T, B_PER_W * T)], idx_v)

    bufs = (buf0, buf1)
    sems = (sem0, sem1)

    def start_gather(j):
        cp = pltpu.make_async_copy(
            emb_hbm.at[idx_v.at[pl.ds(j * CHUNK, CHUNK)]],
            bufs[j % 2],
            sems[j % 2],
        )
        cp.start()
        return cp

    copies = [start_gather(0)]
    for j in range(N_CHUNKS):
        if j + 1 < N_CHUNKS:
            copies.append(start_gather(j + 1))
        copies[j].wait()
        buf = bufs[j % 2]
        pltpu.sync_copy(buf.at[pl.ds(0, T)], out_hbm.at[base_b + BPC * j])
        pltpu.sync_copy(buf.at[pl.ds(T, T)], out_hbm.at[base_b + BPC * j + 1])


def kernel(xb, emb):
    idx = xb.reshape(-1)
    return _gather_rows(emb, idx)
